# baseline (speedup 1.0000x reference)
import functools

import jax
import jax.numpy as jnp
from jax import lax
from jax.experimental import pallas as pl
from jax.experimental.pallas import tpu as pltpu
from jax.experimental.pallas import tpu_sc as plsc

N = 10000
E = 320000
R = 8
DIN = 128
H = 32
DOUT = 64
EPS = 1e-5

NW = 32
BATCH = 128
NB0 = 160
NB_TOT = 16 * NB0
E_PAD = NB_TOT * BATCH
ACC_ROWS = 10112
DUMMY = 10008
RPS = ACC_ROWS // 16
DEGW = 16


def _transform(h, W, root, b, din, dout):
    nblk = 10
    bn = N // nblk

    def body(h_ref, w_ref, root_ref, b_ref, tab_ref, xr_ref):
        hb = h_ref[...]
        parts = [
            lax.dot_general(hb, w_ref[r], (((1,), (0,)), ((), ())),
                            preferred_element_type=jnp.float32)
            for r in range(R)
        ]
        tab_ref[...] = jnp.concatenate(parts, axis=1)
        xr_ref[...] = lax.dot_general(hb, root_ref[...], (((1,), (0,)), ((), ())),
                                      preferred_element_type=jnp.float32) + b_ref[...]

    tab, xr = pl.pallas_call(
        body,
        grid=(nblk,),
        in_specs=[
            pl.BlockSpec((bn, din), lambda i: (i, 0)),
            pl.BlockSpec((R, din, dout), lambda i: (0, 0, 0)),
            pl.BlockSpec((din, dout), lambda i: (0, 0)),
            pl.BlockSpec((1, dout), lambda i: (0, 0)),
        ],
        out_specs=[
            pl.BlockSpec((bn, R * dout), lambda i: (i, 0)),
            pl.BlockSpec((bn, dout), lambda i: (i, 0)),
        ],
        out_shape=[
            jax.ShapeDtypeStruct((N, R * dout), jnp.float32),
            jax.ShapeDtypeStruct((N, dout), jnp.float32),
        ],
    )(h, W, root, b.reshape(1, dout))
    return tab, xr


def _make_sc_edge(tw, with_deg, nb, nb0=NB0):
    mesh = plsc.VectorSubcoreMesh(core_axis_name="c", subcore_axis_name="s")

    out_type = jax.ShapeDtypeStruct((ACC_ROWS, tw), jnp.float32)
    if with_deg:
        out_type = [out_type,
                    jax.ShapeDtypeStruct((ACC_ROWS, DEGW), jnp.float32)]
    scratch = [
        pltpu.VMEM((nb0, BATCH), jnp.int32),
        pltpu.VMEM((nb0, BATCH), jnp.int32),
        pltpu.VMEM((2 * nb, BATCH, tw), jnp.float32),
        pltpu.VMEM_SHARED((ACC_ROWS, tw), jnp.float32),
        [pltpu.SemaphoreType.DMA] * (2 * nb),
        [pltpu.SemaphoreType.DMA] * (2 * nb),
    ]
    if with_deg:
        scratch += [
            pltpu.VMEM((BATCH, DEGW), jnp.float32),
            pltpu.VMEM_SHARED((ACC_ROWS, DEGW), jnp.float32),
            [pltpu.SemaphoreType.DMA] * (2 * nb),
        ]

    def body(*refs):
        if with_deg:
            (table, gidx, didx, ztw, zdeg, ones_h,
             out, out_deg, gv, dv, rows, acc, gsem, ssem,
             ones_v, dacc, dsem) = refs
        else:
            table, gidx, didx, ztw, out, gv, dv, rows, acc, gsem, ssem = refs
        c = lax.axis_index("c")
        s = lax.axis_index("s")
        base = s * nb0
        ngrp = nb0 // (2 * nb)

        def fire(j, slot):
            pltpu.async_copy(table.at[gv.at[j]], rows.at[slot], gsem[slot])

        def process(j, slot):
            pltpu.make_async_copy(table.at[gv.at[j]], rows.at[slot],
                                  gsem[slot]).wait()
            pltpu.async_copy(rows.at[slot], acc.at[dv.at[j]], ssem[slot],
                             add=True)
            if with_deg:
                pltpu.async_copy(ones_v, dacc.at[dv.at[j]], dsem[slot],
                                 add=True)

        def drain(j, slot):
            pltpu.make_async_copy(rows.at[slot], acc.at[dv.at[j]],
                                  ssem[slot]).wait()
            if with_deg:
                pltpu.make_async_copy(ones_v, dacc.at[dv.at[j]],
                                      dsem[slot]).wait()

        def double_group(j0, fire_last):
            for b in range(nb):
                fire(j0 + nb + b, nb + b)
            for b in range(nb):
                process(j0 + b, b)
            for b in range(nb):
                drain(j0 + b, b)
            if fire_last:
                for b in range(nb):
                    fire(j0 + 2 * nb + b, b)
            for b in range(nb):
                process(j0 + nb + b, nb + b)
            for b in range(nb):
                drain(j0 + nb + b, nb + b)

        @pl.when(c == 0)
        def _core0_body():
            pltpu.sync_copy(ztw, acc.at[pl.ds(s * RPS, RPS)])
            if with_deg:
                pltpu.sync_copy(zdeg, dacc.at[pl.ds(s * RPS, RPS)])
                pltpu.sync_copy(ones_h, ones_v)
            pltpu.sync_copy(gidx.at[pl.ds(base, nb0)], gv)
            pltpu.sync_copy(didx.at[pl.ds(base, nb0)], dv)
            plsc.subcore_barrier()

            for b in range(nb):
                fire(b, b)

            def grp(i, carry):
                double_group(i * 2 * nb, True)
                return carry

            lax.fori_loop(0, ngrp - 1, grp, 0)
            double_group((ngrp - 1) * 2 * nb, False)

            plsc.subcore_barrier()
            pltpu.sync_copy(acc.at[pl.ds(s * RPS, RPS)],
                            out.at[pl.ds(s * RPS, RPS)])
            if with_deg:
                pltpu.sync_copy(dacc.at[pl.ds(s * RPS, RPS)],
                                out_deg.at[pl.ds(s * RPS, RPS)])

    return pl.kernel(body, out_type=out_type, mesh=mesh, scratch_types=scratch,
                     compiler_params=pltpu.CompilerParams(use_tc_tiling_on_sc=False))


def _bn_first(p, dp, xr, g, beta):
    def body(p_ref, dp_ref, xr_ref, g_ref, beta_ref, h_ref, dinv_ref):
        s = p_ref[0:N, :]
        deg = dp_ref[0:N, 0:1]
        dinv = 1.0 / jnp.maximum(deg, 1.0)
        y = s * dinv + xr_ref[...]
        m = jnp.mean(y, axis=0, keepdims=True)
        yc = y - m
        v = jnp.mean(yc * yc, axis=0, keepdims=True)
        h = g_ref[...] * yc * lax.rsqrt(v + EPS) + beta_ref[...]
        h_ref[...] = jnp.maximum(h, 0.0)
        dinv_ref[...] = dinv

    return pl.pallas_call(
        body,
        out_shape=[
            jax.ShapeDtypeStruct((N, H), jnp.float32),
            jax.ShapeDtypeStruct((N, 1), jnp.float32),
        ],
    )(p, dp, xr, g.reshape(1, H), beta.reshape(1, H))


def _bn_mid(p, xr, dinv, g, beta, dout):
    def body(p_ref, xr_ref, dinv_ref, g_ref, beta_ref, h_ref):
        s = p_ref[0:N, :]
        y = s * dinv_ref[...] + xr_ref[...]
        m = jnp.mean(y, axis=0, keepdims=True)
        yc = y - m
        v = jnp.mean(yc * yc, axis=0, keepdims=True)
        h = g_ref[...] * yc * lax.rsqrt(v + EPS) + beta_ref[...]
        h_ref[...] = jnp.maximum(h, 0.0)

    return pl.pallas_call(
        body,
        out_shape=jax.ShapeDtypeStruct((N, dout), jnp.float32),
    )(p, xr, dinv, g.reshape(1, dout), beta.reshape(1, dout))


def _bn_final(p, xr, dinv, g, beta, linW, linb):
    def body(p_ref, xr_ref, dinv_ref, g_ref, beta_ref, lw_ref, lb_ref, o_ref):
        s = p_ref[0:N, :]
        y = s * dinv_ref[...] + xr_ref[...]
        m = jnp.mean(y, axis=0, keepdims=True)
        yc = y - m
        v = jnp.mean(yc * yc, axis=0, keepdims=True)
        h = g_ref[...] * yc * lax.rsqrt(v + EPS) + beta_ref[...]
        o_ref[...] = lax.dot_general(h, lw_ref[...], (((1,), (0,)), ((), ())),
                                     preferred_element_type=jnp.float32) + lb_ref[...]

    return pl.pallas_call(
        body,
        out_shape=jax.ShapeDtypeStruct((N, 2), jnp.float32),
    )(p, xr, dinv, g.reshape(1, DOUT), beta.reshape(1, DOUT), linW, linb.reshape(1, 2))


_sc_edge_deg = _make_sc_edge(H, True, 4)
_sc_edge_h = _make_sc_edge(H, False, 4)
_sc_edge_o = _make_sc_edge(DOUT, False, 2)


def kernel(x, edge_index, edge_type,
           W1, root1, b1, g1, beta1,
           W2, root2, b2, g2, beta2,
           W3, root3, b3, g3, beta3,
           W4, root4, b4, g4, beta4,
           linW, linb):
    src = edge_index[0]
    dst = edge_index[1]
    gidx = src * R + edge_type
    npad = NB_TOT * BATCH - E
    pad_g = jnp.zeros((npad,), jnp.int32)
    pad_d = (N + jnp.arange(npad, dtype=jnp.int32) % (ACC_ROWS - N))
    gidx2 = jnp.concatenate([gidx.astype(jnp.int32), pad_g]).reshape(NB_TOT, BATCH)
    didx2 = jnp.concatenate([dst.astype(jnp.int32), pad_d]).reshape(NB_TOT, BATCH)

    z32 = jnp.zeros((RPS, H), jnp.float32)
    z64 = jnp.zeros((RPS, DOUT), jnp.float32)
    zdeg = jnp.zeros((RPS, DEGW), jnp.float32)
    ones16 = jnp.ones((BATCH, DEGW), jnp.float32)

    tab1, xr1 = _transform(x, W1, root1, b1, DIN, H)
    p1, dp1 = _sc_edge_deg(tab1.reshape(N * R, H), gidx2, didx2,
                           z32, zdeg, ones16)
    h1, dinv = _bn_first(p1, dp1, xr1, g1, beta1)

    tab2, xr2 = _transform(h1, W2, root2, b2, H, H)
    p2 = _sc_edge_h(tab2.reshape(N * R, H), gidx2, didx2, z32)
    h2 = _bn_mid(p2, xr2, dinv, g2, beta2, H)

    tab3, xr3 = _transform(h2, W3, root3, b3, H, H)
    p3 = _sc_edge_h(tab3.reshape(N * R, H), gidx2, didx2, z32)
    h3 = _bn_mid(p3, xr3, dinv, g3, beta3, H)

    tab4, xr4 = _transform(h3, W4, root4, b4, H, DOUT)
    p4 = _sc_edge_o(tab4.reshape(N * R, DOUT), gidx2, didx2, z64)
    return _bn_final(p4, xr4, dinv, g4, beta4, linW, linb)


# speedup vs baseline: 1.1698x; 1.1698x over previous
import functools

import jax
import jax.numpy as jnp
from jax import lax
from jax.experimental import pallas as pl
from jax.experimental.pallas import tpu as pltpu
from jax.experimental.pallas import tpu_sc as plsc

N = 10000
E = 320000
R = 8
DIN = 128
H = 32
DOUT = 64
EPS = 1e-5

NW = 32
BATCH = 128
NB0 = 144
NB1 = 16
NB_TOT = 16 * NB0 + 15 * NB1 + NB0
E_PAD = 16 * (NB0 + NB1) * BATCH
ACC_ROWS = 10112
DUMMY = 10008
RPS = ACC_ROWS // 16
DEGW = 16


def _transform(h, W, root, b, din, dout):
    nblk = 10
    bn = N // nblk

    def body(h_ref, w_ref, root_ref, b_ref, tab_ref, xr_ref):
        hb = h_ref[...]
        parts = [
            lax.dot_general(hb, w_ref[r], (((1,), (0,)), ((), ())),
                            preferred_element_type=jnp.float32)
            for r in range(R)
        ]
        tab_ref[...] = jnp.concatenate(parts, axis=1)
        xr_ref[...] = lax.dot_general(hb, root_ref[...], (((1,), (0,)), ((), ())),
                                      preferred_element_type=jnp.float32) + b_ref[...]

    tab, xr = pl.pallas_call(
        body,
        grid=(nblk,),
        in_specs=[
            pl.BlockSpec((bn, din), lambda i: (i, 0)),
            pl.BlockSpec((R, din, dout), lambda i: (0, 0, 0)),
            pl.BlockSpec((din, dout), lambda i: (0, 0)),
            pl.BlockSpec((1, dout), lambda i: (0, 0)),
        ],
        out_specs=[
            pl.BlockSpec((bn, R * dout), lambda i: (i, 0)),
            pl.BlockSpec((bn, dout), lambda i: (i, 0)),
        ],
        out_shape=[
            jax.ShapeDtypeStruct((N, R * dout), jnp.float32),
            jax.ShapeDtypeStruct((N, dout), jnp.float32),
        ],
    )(h, W, root, b.reshape(1, dout))
    return tab, xr


def _make_sc_edge(tw, with_deg, nb, nb0=NB0, nb1=NB1):
    mesh = plsc.VectorSubcoreMesh(core_axis_name="c", subcore_axis_name="s")

    out_type = jax.ShapeDtypeStruct((2 * ACC_ROWS, tw), jnp.float32)
    if with_deg:
        out_type = [out_type,
                    jax.ShapeDtypeStruct((2 * ACC_ROWS, DEGW), jnp.float32)]
    scratch = [
        pltpu.VMEM((nb0, BATCH), jnp.int32),
        pltpu.VMEM((nb0, BATCH), jnp.int32),
        pltpu.VMEM((2 * nb, BATCH, tw), jnp.float32),
        pltpu.VMEM_SHARED((ACC_ROWS, tw), jnp.float32),
        [pltpu.SemaphoreType.DMA] * (2 * nb),
        [pltpu.SemaphoreType.DMA] * (2 * nb),
    ]
    if with_deg:
        scratch += [
            pltpu.VMEM((BATCH, DEGW), jnp.float32),
            pltpu.VMEM_SHARED((ACC_ROWS, DEGW), jnp.float32),
            [pltpu.SemaphoreType.DMA] * (2 * nb),
        ]

    def body(*refs):
        if with_deg:
            (table, gidx, didx, ztw, zdeg, ones_h,
             out, out_deg, gv, dv, rows, acc, gsem, ssem,
             ones_v, dacc, dsem) = refs
        else:
            table, gidx, didx, ztw, out, gv, dv, rows, acc, gsem, ssem = refs
        c = lax.axis_index("c")
        s = lax.axis_index("s")
        base = jnp.where(c == 0, s * nb0, 16 * nb0 + s * nb1)
        ngrp = jnp.where(c == 0, nb0 // (2 * nb), nb1 // (2 * nb))

        pltpu.sync_copy(ztw, acc.at[pl.ds(s * RPS, RPS)])
        if with_deg:
            pltpu.sync_copy(zdeg, dacc.at[pl.ds(s * RPS, RPS)])
            pltpu.sync_copy(ones_h, ones_v)
        pltpu.sync_copy(gidx.at[pl.ds(base, nb0)], gv)
        pltpu.sync_copy(didx.at[pl.ds(base, nb0)], dv)
        plsc.subcore_barrier()

        def fire(j, slot):
            pltpu.async_copy(table.at[gv.at[j]], rows.at[slot], gsem[slot])

        def process(j, slot):
            pltpu.make_async_copy(table.at[gv.at[j]], rows.at[slot],
                                  gsem[slot]).wait()
            pltpu.async_copy(rows.at[slot], acc.at[dv.at[j]], ssem[slot],
                             add=True)
            if with_deg:
                pltpu.async_copy(ones_v, dacc.at[dv.at[j]], dsem[slot],
                                 add=True)

        def drain(j, slot):
            pltpu.make_async_copy(rows.at[slot], acc.at[dv.at[j]],
                                  ssem[slot]).wait()
            if with_deg:
                pltpu.make_async_copy(ones_v, dacc.at[dv.at[j]],
                                      dsem[slot]).wait()

        def double_group(j0, fire_last):
            for b in range(nb):
                fire(j0 + nb + b, nb + b)
            for b in range(nb):
                process(j0 + b, b)
            for b in range(nb):
                drain(j0 + b, b)
            if fire_last:
                for b in range(nb):
                    fire(j0 + 2 * nb + b, b)
            for b in range(nb):
                process(j0 + nb + b, nb + b)
            for b in range(nb):
                drain(j0 + nb + b, nb + b)

        for b in range(nb):
            fire(b, b)

        def grp(i, carry):
            double_group(i * 2 * nb, True)
            return carry

        lax.fori_loop(0, ngrp - 1, grp, 0)
        double_group((ngrp - 1) * 2 * nb, False)

        plsc.subcore_barrier()
        pltpu.sync_copy(acc.at[pl.ds(s * RPS, RPS)],
                        out.at[pl.ds(c * ACC_ROWS + s * RPS, RPS)])
        if with_deg:
            pltpu.sync_copy(dacc.at[pl.ds(s * RPS, RPS)],
                            out_deg.at[pl.ds(c * ACC_ROWS + s * RPS, RPS)])

    return pl.kernel(body, out_type=out_type, mesh=mesh, scratch_types=scratch,
                     compiler_params=pltpu.CompilerParams(use_tc_tiling_on_sc=False))


def _bn_first(p, dp, xr, g, beta):
    def body(p_ref, dp_ref, xr_ref, g_ref, beta_ref, h_ref, dinv_ref):
        s = p_ref[0:N, :] + p_ref[ACC_ROWS:ACC_ROWS + N, :]
        deg = dp_ref[0:N, 0:1] + dp_ref[ACC_ROWS:ACC_ROWS + N, 0:1]
        dinv = 1.0 / jnp.maximum(deg, 1.0)
        y = s * dinv + xr_ref[...]
        m = jnp.mean(y, axis=0, keepdims=True)
        yc = y - m
        v = jnp.mean(yc * yc, axis=0, keepdims=True)
        h = g_ref[...] * yc * lax.rsqrt(v + EPS) + beta_ref[...]
        h_ref[...] = jnp.maximum(h, 0.0)
        dinv_ref[...] = dinv

    return pl.pallas_call(
        body,
        out_shape=[
            jax.ShapeDtypeStruct((N, H), jnp.float32),
            jax.ShapeDtypeStruct((N, 1), jnp.float32),
        ],
    )(p, dp, xr, g.reshape(1, H), beta.reshape(1, H))


def _bn_mid(p, xr, dinv, g, beta, dout):
    def body(p_ref, xr_ref, dinv_ref, g_ref, beta_ref, h_ref):
        s = p_ref[0:N, :] + p_ref[ACC_ROWS:ACC_ROWS + N, :]
        y = s * dinv_ref[...] + xr_ref[...]
        m = jnp.mean(y, axis=0, keepdims=True)
        yc = y - m
        v = jnp.mean(yc * yc, axis=0, keepdims=True)
        h = g_ref[...] * yc * lax.rsqrt(v + EPS) + beta_ref[...]
        h_ref[...] = jnp.maximum(h, 0.0)

    return pl.pallas_call(
        body,
        out_shape=jax.ShapeDtypeStruct((N, dout), jnp.float32),
    )(p, xr, dinv, g.reshape(1, dout), beta.reshape(1, dout))


def _bn_final(p, xr, dinv, g, beta, linW, linb):
    def body(p_ref, xr_ref, dinv_ref, g_ref, beta_ref, lw_ref, lb_ref, o_ref):
        s = p_ref[0:N, :] + p_ref[ACC_ROWS:ACC_ROWS + N, :]
        y = s * dinv_ref[...] + xr_ref[...]
        m = jnp.mean(y, axis=0, keepdims=True)
        yc = y - m
        v = jnp.mean(yc * yc, axis=0, keepdims=True)
        h = g_ref[...] * yc * lax.rsqrt(v + EPS) + beta_ref[...]
        o_ref[...] = lax.dot_general(h, lw_ref[...], (((1,), (0,)), ((), ())),
                                     preferred_element_type=jnp.float32) + lb_ref[...]

    return pl.pallas_call(
        body,
        out_shape=jax.ShapeDtypeStruct((N, 2), jnp.float32),
    )(p, xr, dinv, g.reshape(1, DOUT), beta.reshape(1, DOUT), linW, linb.reshape(1, 2))


_sc_edge_deg = _make_sc_edge(H, True, 4)
_sc_edge_h = _make_sc_edge(H, False, 4)
_sc_edge_o = _make_sc_edge(DOUT, False, 2)


def kernel(x, edge_index, edge_type,
           W1, root1, b1, g1, beta1,
           W2, root2, b2, g2, beta2,
           W3, root3, b3, g3, beta3,
           W4, root4, b4, g4, beta4,
           linW, linb):
    src = edge_index[0]
    dst = edge_index[1]
    gidx = src * R + edge_type
    npad = NB_TOT * BATCH - E
    pad_g = jnp.zeros((npad,), jnp.int32)
    pad_d = (N + jnp.arange(npad, dtype=jnp.int32) % (ACC_ROWS - N))
    gidx2 = jnp.concatenate([gidx.astype(jnp.int32), pad_g]).reshape(NB_TOT, BATCH)
    didx2 = jnp.concatenate([dst.astype(jnp.int32), pad_d]).reshape(NB_TOT, BATCH)

    z32 = jnp.zeros((RPS, H), jnp.float32)
    z64 = jnp.zeros((RPS, DOUT), jnp.float32)
    zdeg = jnp.zeros((RPS, DEGW), jnp.float32)
    ones16 = jnp.ones((BATCH, DEGW), jnp.float32)

    tab1, xr1 = _transform(x, W1, root1, b1, DIN, H)
    p1, dp1 = _sc_edge_deg(tab1.reshape(N * R, H), gidx2, didx2,
                           z32, zdeg, ones16)
    h1, dinv = _bn_first(p1, dp1, xr1, g1, beta1)

    tab2, xr2 = _transform(h1, W2, root2, b2, H, H)
    p2 = _sc_edge_h(tab2.reshape(N * R, H), gidx2, didx2, z32)
    h2 = _bn_mid(p2, xr2, dinv, g2, beta2, H)

    tab3, xr3 = _transform(h2, W3, root3, b3, H, H)
    p3 = _sc_edge_h(tab3.reshape(N * R, H), gidx2, didx2, z32)
    h3 = _bn_mid(p3, xr3, dinv, g3, beta3, H)

    tab4, xr4 = _transform(h3, W4, root4, b4, H, DOUT)
    p4 = _sc_edge_o(tab4.reshape(N * R, DOUT), gidx2, didx2, z64)
    return _bn_final(p4, xr4, dinv, g4, beta4, linW, linb)


# speedup vs baseline: 1.1797x; 1.0085x over previous
import functools

import jax
import jax.numpy as jnp
from jax import lax
from jax.experimental import pallas as pl
from jax.experimental.pallas import tpu as pltpu
from jax.experimental.pallas import tpu_sc as plsc

N = 10000
E = 320000
R = 8
DIN = 128
H = 32
DOUT = 64
EPS = 1e-5

NW = 32
BATCH = 128
NB0 = 152
NB1 = 8
NB_TOT = 16 * NB0 + 15 * NB1 + NB0
E_PAD = 16 * (NB0 + NB1) * BATCH
ACC_ROWS = 10112
DUMMY = 10008
RPS = ACC_ROWS // 16
DEGW = 16


def _transform(h, W, root, b, din, dout):
    nblk = 10
    bn = N // nblk

    def body(h_ref, w_ref, root_ref, b_ref, tab_ref, xr_ref):
        hb = h_ref[...]
        parts = [
            lax.dot_general(hb, w_ref[r], (((1,), (0,)), ((), ())),
                            preferred_element_type=jnp.float32)
            for r in range(R)
        ]
        tab_ref[...] = jnp.concatenate(parts, axis=1)
        xr_ref[...] = lax.dot_general(hb, root_ref[...], (((1,), (0,)), ((), ())),
                                      preferred_element_type=jnp.float32) + b_ref[...]

    tab, xr = pl.pallas_call(
        body,
        grid=(nblk,),
        in_specs=[
            pl.BlockSpec((bn, din), lambda i: (i, 0)),
            pl.BlockSpec((R, din, dout), lambda i: (0, 0, 0)),
            pl.BlockSpec((din, dout), lambda i: (0, 0)),
            pl.BlockSpec((1, dout), lambda i: (0, 0)),
        ],
        out_specs=[
            pl.BlockSpec((bn, R * dout), lambda i: (i, 0)),
            pl.BlockSpec((bn, dout), lambda i: (i, 0)),
        ],
        out_shape=[
            jax.ShapeDtypeStruct((N, R * dout), jnp.float32),
            jax.ShapeDtypeStruct((N, dout), jnp.float32),
        ],
    )(h, W, root, b.reshape(1, dout))
    return tab, xr


def _make_sc_edge(tw, with_deg, nb, nb0=NB0, nb1=NB1):
    mesh = plsc.VectorSubcoreMesh(core_axis_name="c", subcore_axis_name="s")

    out_type = jax.ShapeDtypeStruct((2 * ACC_ROWS, tw), jnp.float32)
    if with_deg:
        out_type = [out_type,
                    jax.ShapeDtypeStruct((2 * ACC_ROWS, DEGW), jnp.float32)]
    scratch = [
        pltpu.VMEM((nb0, BATCH), jnp.int32),
        pltpu.VMEM((nb0, BATCH), jnp.int32),
        pltpu.VMEM((2 * nb, BATCH, tw), jnp.float32),
        pltpu.VMEM_SHARED((ACC_ROWS, tw), jnp.float32),
        [pltpu.SemaphoreType.DMA] * (2 * nb),
        [pltpu.SemaphoreType.DMA] * (2 * nb),
    ]
    if with_deg:
        scratch += [
            pltpu.VMEM((BATCH, DEGW), jnp.float32),
            pltpu.VMEM_SHARED((ACC_ROWS, DEGW), jnp.float32),
            [pltpu.SemaphoreType.DMA] * (2 * nb),
        ]

    def body(*refs):
        if with_deg:
            (table, gidx, didx, ztw, zdeg, ones_h,
             out, out_deg, gv, dv, rows, acc, gsem, ssem,
             ones_v, dacc, dsem) = refs
        else:
            table, gidx, didx, ztw, out, gv, dv, rows, acc, gsem, ssem = refs
        c = lax.axis_index("c")
        s = lax.axis_index("s")
        base = jnp.where(c == 0, s * nb0, 16 * nb0 + s * nb1)
        ngrp = jnp.where(c == 0, nb0 // (2 * nb), nb1 // (2 * nb))

        pltpu.sync_copy(ztw, acc.at[pl.ds(s * RPS, RPS)])
        if with_deg:
            pltpu.sync_copy(zdeg, dacc.at[pl.ds(s * RPS, RPS)])
            pltpu.sync_copy(ones_h, ones_v)
        pltpu.sync_copy(gidx.at[pl.ds(base, nb0)], gv)
        pltpu.sync_copy(didx.at[pl.ds(base, nb0)], dv)
        plsc.subcore_barrier()

        def fire(j, slot):
            pltpu.async_copy(table.at[gv.at[j]], rows.at[slot], gsem[slot])

        def process(j, slot):
            pltpu.make_async_copy(table.at[gv.at[j]], rows.at[slot],
                                  gsem[slot]).wait()
            pltpu.async_copy(rows.at[slot], acc.at[dv.at[j]], ssem[slot],
                             add=True)
            if with_deg:
                pltpu.async_copy(ones_v, dacc.at[dv.at[j]], dsem[slot],
                                 add=True)

        def drain(j, slot):
            pltpu.make_async_copy(rows.at[slot], acc.at[dv.at[j]],
                                  ssem[slot]).wait()
            if with_deg:
                pltpu.make_async_copy(ones_v, dacc.at[dv.at[j]],
                                      dsem[slot]).wait()

        def double_group(j0, fire_last):
            for b in range(nb):
                fire(j0 + nb + b, nb + b)
            for b in range(nb):
                process(j0 + b, b)
            for b in range(nb):
                drain(j0 + b, b)
            if fire_last:
                for b in range(nb):
                    fire(j0 + 2 * nb + b, b)
            for b in range(nb):
                process(j0 + nb + b, nb + b)
            for b in range(nb):
                drain(j0 + nb + b, nb + b)

        for b in range(nb):
            fire(b, b)

        def grp(i, carry):
            double_group(i * 2 * nb, True)
            return carry

        lax.fori_loop(0, ngrp - 1, grp, 0)
        double_group((ngrp - 1) * 2 * nb, False)

        plsc.subcore_barrier()
        pltpu.sync_copy(acc.at[pl.ds(s * RPS, RPS)],
                        out.at[pl.ds(c * ACC_ROWS + s * RPS, RPS)])
        if with_deg:
            pltpu.sync_copy(dacc.at[pl.ds(s * RPS, RPS)],
                            out_deg.at[pl.ds(c * ACC_ROWS + s * RPS, RPS)])

    return pl.kernel(body, out_type=out_type, mesh=mesh, scratch_types=scratch,
                     compiler_params=pltpu.CompilerParams(use_tc_tiling_on_sc=False))


def _bn_first(p, dp, xr, g, beta):
    def body(p_ref, dp_ref, xr_ref, g_ref, beta_ref, h_ref, dinv_ref):
        s = p_ref[0:N, :] + p_ref[ACC_ROWS:ACC_ROWS + N, :]
        deg = dp_ref[0:N, 0:1] + dp_ref[ACC_ROWS:ACC_ROWS + N, 0:1]
        dinv = 1.0 / jnp.maximum(deg, 1.0)
        y = s * dinv + xr_ref[...]
        m = jnp.mean(y, axis=0, keepdims=True)
        yc = y - m
        v = jnp.mean(yc * yc, axis=0, keepdims=True)
        h = g_ref[...] * yc * lax.rsqrt(v + EPS) + beta_ref[...]
        h_ref[...] = jnp.maximum(h, 0.0)
        dinv_ref[...] = dinv

    return pl.pallas_call(
        body,
        out_shape=[
            jax.ShapeDtypeStruct((N, H), jnp.float32),
            jax.ShapeDtypeStruct((N, 1), jnp.float32),
        ],
    )(p, dp, xr, g.reshape(1, H), beta.reshape(1, H))


def _bn_mid(p, xr, dinv, g, beta, dout):
    def body(p_ref, xr_ref, dinv_ref, g_ref, beta_ref, h_ref):
        s = p_ref[0:N, :] + p_ref[ACC_ROWS:ACC_ROWS + N, :]
        y = s * dinv_ref[...] + xr_ref[...]
        m = jnp.mean(y, axis=0, keepdims=True)
        yc = y - m
        v = jnp.mean(yc * yc, axis=0, keepdims=True)
        h = g_ref[...] * yc * lax.rsqrt(v + EPS) + beta_ref[...]
        h_ref[...] = jnp.maximum(h, 0.0)

    return pl.pallas_call(
        body,
        out_shape=jax.ShapeDtypeStruct((N, dout), jnp.float32),
    )(p, xr, dinv, g.reshape(1, dout), beta.reshape(1, dout))


def _bn_final(p, xr, dinv, g, beta, linW, linb):
    def body(p_ref, xr_ref, dinv_ref, g_ref, beta_ref, lw_ref, lb_ref, o_ref):
        s = p_ref[0:N, :] + p_ref[ACC_ROWS:ACC_ROWS + N, :]
        y = s * dinv_ref[...] + xr_ref[...]
        m = jnp.mean(y, axis=0, keepdims=True)
        yc = y - m
        v = jnp.mean(yc * yc, axis=0, keepdims=True)
        h = g_ref[...] * yc * lax.rsqrt(v + EPS) + beta_ref[...]
        o_ref[...] = lax.dot_general(h, lw_ref[...], (((1,), (0,)), ((), ())),
                                     preferred_element_type=jnp.float32) + lb_ref[...]

    return pl.pallas_call(
        body,
        out_shape=jax.ShapeDtypeStruct((N, 2), jnp.float32),
    )(p, xr, dinv, g.reshape(1, DOUT), beta.reshape(1, DOUT), linW, linb.reshape(1, 2))


_sc_edge_deg = _make_sc_edge(H, True, 4)
_sc_edge_h = _make_sc_edge(H, False, 4)
_sc_edge_o = _make_sc_edge(DOUT, False, 2)


def kernel(x, edge_index, edge_type,
           W1, root1, b1, g1, beta1,
           W2, root2, b2, g2, beta2,
           W3, root3, b3, g3, beta3,
           W4, root4, b4, g4, beta4,
           linW, linb):
    src = edge_index[0]
    dst = edge_index[1]
    gidx = src * R + edge_type
    npad = NB_TOT * BATCH - E
    pad_g = jnp.zeros((npad,), jnp.int32)
    pad_d = (N + jnp.arange(npad, dtype=jnp.int32) % (ACC_ROWS - N))
    gidx2 = jnp.concatenate([gidx.astype(jnp.int32), pad_g]).reshape(NB_TOT, BATCH)
    didx2 = jnp.concatenate([dst.astype(jnp.int32), pad_d]).reshape(NB_TOT, BATCH)

    z32 = jnp.zeros((RPS, H), jnp.float32)
    z64 = jnp.zeros((RPS, DOUT), jnp.float32)
    zdeg = jnp.zeros((RPS, DEGW), jnp.float32)
    ones16 = jnp.ones((BATCH, DEGW), jnp.float32)

    tab1, xr1 = _transform(x, W1, root1, b1, DIN, H)
    p1, dp1 = _sc_edge_deg(tab1.reshape(N * R, H), gidx2, didx2,
                           z32, zdeg, ones16)
    h1, dinv = _bn_first(p1, dp1, xr1, g1, beta1)

    tab2, xr2 = _transform(h1, W2, root2, b2, H, H)
    p2 = _sc_edge_h(tab2.reshape(N * R, H), gidx2, didx2, z32)
    h2 = _bn_mid(p2, xr2, dinv, g2, beta2, H)

    tab3, xr3 = _transform(h2, W3, root3, b3, H, H)
    p3 = _sc_edge_h(tab3.reshape(N * R, H), gidx2, didx2, z32)
    h3 = _bn_mid(p3, xr3, dinv, g3, beta3, H)

    tab4, xr4 = _transform(h3, W4, root4, b4, H, DOUT)
    p4 = _sc_edge_o(tab4.reshape(N * R, DOUT), gidx2, didx2, z64)
    return _bn_final(p4, xr4, dinv, g4, beta4, linW, linb)
